# baseline (speedup 1.0000x reference)
import functools
import math

import jax
import jax.numpy as jnp
from jax import lax
from jax.experimental import pallas as pl
from jax.experimental.pallas import tpu as pltpu
from jax.experimental.pallas import tpu_sc as plsc

D = 128
SCALE = math.sqrt(128.0)

_info = plsc.get_sparse_core_info()
_NC = _info.num_cores
_NS = _info.num_subcores
_NW = _NC * _NS
_L = _info.num_lanes

CH = 128
NBUF = 5


@functools.lru_cache(maxsize=None)
def _emb_call(n_chunks):
    n = _NW * n_chunks * CH
    mesh = plsc.VectorSubcoreMesh(core_axis_name="c", subcore_axis_name="s")

    @functools.partial(
        pl.kernel,
        mesh=mesh,
        out_type=jax.ShapeDtypeStruct((n, D), jnp.float32),
        scratch_types=(
            [pltpu.VMEM((n_chunks, CH), jnp.int32)]
            + [pltpu.VMEM((CH, D), jnp.float32)] * NBUF
            + [pltpu.SemaphoreType.DMA] * (2 * NBUF)
        ),
        compiler_params=pltpu.CompilerParams(skip_device_barrier=True),
    )
    def k(idx_hbm, lut_hbm, out_hbm, idx_v, *rest):
        rows = rest[:NBUF]
        gsem = rest[NBUF:2 * NBUF]
        ssem = rest[2 * NBUF:3 * NBUF]

        wid = lax.axis_index("s") * _NC + lax.axis_index("c")
        pltpu.sync_copy(idx_hbm.at[wid], idx_v)
        base = wid * (n_chunks * CH)

        def start_gather(b, j):
            pltpu.async_copy(lut_hbm.at[idx_v.at[j]], rows[b], gsem[b])

        def wait_gather(b):
            pltpu.make_async_copy(lut_hbm.at[idx_v.at[0]], rows[b],
                                  gsem[b]).wait()

        def start_store(b, j):
            pltpu.async_copy(rows[b], out_hbm.at[pl.ds(base + j * CH, CH)],
                             ssem[b])

        def wait_store(b):
            pltpu.make_async_copy(rows[b], out_hbm.at[pl.ds(base, CH)],
                                  ssem[b]).wait()

        for b in range(NBUF):
            start_gather(b, b)

        def group_body(g, carry):
            for b in range(NBUF):
                j = g * NBUF + b
                wait_gather(b)

                def scale_body(r, c2, _b=b):
                    for u in range(4):
                        rr = r * 4 + u
                        for c in range(D // _L):
                            sl = pl.ds(c * _L, _L)
                            rows[_b][rr, sl] = rows[_b][rr, sl] * SCALE
                    return c2

                lax.fori_loop(0, CH // 4, scale_body, 0)
                start_store(b, j)

                q = j + NBUF - 1
                pb = (b - 1) % NBUF

                @pl.when(jnp.logical_and(q >= NBUF, q < n_chunks))
                def _():
                    wait_store(pb)
                    start_gather(pb, q)

            return carry

        lax.fori_loop(0, n_chunks // NBUF, group_body, 0)
        for b in range(NBUF):
            wait_store(b)

    return k


def kernel(x, lut):
    b, h = x.shape
    n = b * h
    n_chunks = n // (_NW * CH)
    idx = jnp.transpose(x).reshape(_NW, n_chunks, CH)
    out = _emb_call(n_chunks)(idx, lut)
    return out.reshape(h, b, D).transpose(1, 0, 2)


# speedup vs baseline: 1.0015x; 1.0015x over previous
import functools
import math

import jax
import jax.numpy as jnp
from jax import lax
from jax.experimental import pallas as pl
from jax.experimental.pallas import tpu as pltpu
from jax.experimental.pallas import tpu_sc as plsc

D = 128
SCALE = math.sqrt(128.0)

_info = plsc.get_sparse_core_info()
_NC = _info.num_cores
_NS = _info.num_subcores
_NW = _NC * _NS
_L = _info.num_lanes

CH = 128
NBUF = 5


@functools.lru_cache(maxsize=None)
def _emb_call(n_chunks):
    n = _NW * n_chunks * CH
    mesh = plsc.VectorSubcoreMesh(core_axis_name="c", subcore_axis_name="s")

    @functools.partial(
        pl.kernel,
        mesh=mesh,
        out_type=jax.ShapeDtypeStruct((n, D), jnp.float32),
        scratch_types=(
            [pltpu.VMEM((n_chunks, CH), jnp.int32)]
            + [pltpu.VMEM((CH, D), jnp.float32)] * NBUF
            + [pltpu.SemaphoreType.DMA] * (2 * NBUF)
        ),
    )
    def k(idx_hbm, lut_hbm, out_hbm, idx_v, *rest):
        rows = rest[:NBUF]
        gsem = rest[NBUF:2 * NBUF]
        ssem = rest[2 * NBUF:3 * NBUF]

        wid = lax.axis_index("s") * _NC + lax.axis_index("c")
        pltpu.sync_copy(idx_hbm.at[wid], idx_v)
        base = wid * (n_chunks * CH)

        def start_gather(b, j):
            pltpu.async_copy(lut_hbm.at[idx_v.at[j]], rows[b], gsem[b])

        def wait_gather(b):
            pltpu.make_async_copy(lut_hbm.at[idx_v.at[0]], rows[b],
                                  gsem[b]).wait()

        def start_store(b, j):
            pltpu.async_copy(rows[b], out_hbm.at[pl.ds(base + j * CH, CH)],
                             ssem[b])

        def wait_store(b):
            pltpu.make_async_copy(rows[b], out_hbm.at[pl.ds(base, CH)],
                                  ssem[b]).wait()

        for b in range(NBUF):
            start_gather(b, b)

        def group_body(g, carry):
            for b in range(NBUF):
                j = g * NBUF + b
                wait_gather(b)

                def scale_body(r, c2, _b=b):
                    for u in range(4):
                        rr = r * 4 + u
                        for c in range(D // _L):
                            sl = pl.ds(c * _L, _L)
                            rows[_b][rr, sl] = rows[_b][rr, sl] * SCALE
                    return c2

                lax.fori_loop(0, CH // 4, scale_body, 0)
                start_store(b, j)

                q = j + NBUF - 1
                pb = (b - 1) % NBUF

                @pl.when(jnp.logical_and(q >= NBUF, q < n_chunks))
                def _():
                    wait_store(pb)
                    start_gather(pb, q)

            return carry

        lax.fori_loop(0, n_chunks // NBUF, group_body, 0)
        for b in range(NBUF):
            wait_store(b)

    return k


def kernel(x, lut):
    b, h = x.shape
    n = b * h
    n_chunks = n // (_NW * CH)
    idx = jnp.transpose(x).reshape(_NW, n_chunks, CH)
    out = _emb_call(n_chunks)(idx, lut)
    return out.reshape(h, b, D).transpose(1, 0, 2)
